# baseline (speedup 1.0000x reference)
import functools

import jax
import jax.numpy as jnp
from jax import lax
from jax.experimental import pallas as pl
from jax.experimental.pallas import tpu as pltpu
from jax.experimental.pallas import tpu_sc as plsc

_FIELDS = 26
_K = 32
_NC = 2
_NS = 16


_CHUNK = 128
_NBUF = 4


def _make_sc_gather(n_idx, k_dim):
    nw = _NC * _NS
    assert n_idx % nw == 0
    b_per_w = n_idx // nw
    assert b_per_w % _CHUNK == 0
    n_chunks = b_per_w // _CHUNK
    mesh = plsc.VectorSubcoreMesh(core_axis_name="c", subcore_axis_name="s")

    @functools.partial(
        pl.kernel,
        mesh=mesh,
        compiler_params=pltpu.CompilerParams(use_tc_tiling_on_sc=False),
        out_type=[
            jax.ShapeDtypeStruct((n_idx, k_dim), jnp.float32),
            jax.ShapeDtypeStruct((n_idx,), jnp.float32),
        ],
        scratch_types=[
            pltpu.VMEM((n_chunks, _CHUNK), jnp.int32),
            [pltpu.VMEM((_CHUNK, k_dim), jnp.float32) for _ in range(_NBUF)],
            [pltpu.VMEM((_CHUNK,), jnp.float32) for _ in range(_NBUF)],
            [pltpu.SemaphoreType.DMA for _ in range(_NBUF)],
            [pltpu.SemaphoreType.DMA for _ in range(_NBUF)],
        ],
    )
    def sc_gather(idx_hbm, v_hbm, w_hbm, emb_out, w_out,
                  idx_v, rows_v, wrows_v, sem_v, sem_w):
        wid = lax.axis_index("s") * _NC + lax.axis_index("c")
        base = wid * b_per_w
        pltpu.sync_copy(
            idx_hbm.at[pl.ds(wid * n_chunks, n_chunks)], idx_v)
        cp_v = [None] * _NBUF
        cp_w = [None] * _NBUF
        for c in range(n_chunks + _NBUF - 1):
            if c < n_chunks:
                b = c % _NBUF
                cp_v[b] = pltpu.async_copy(
                    v_hbm.at[idx_v.at[c]], rows_v[b], sem_v[b])
                cp_w[b] = pltpu.async_copy(
                    w_hbm.at[idx_v.at[c]], wrows_v[b], sem_w[b])
            if c >= _NBUF - 1:
                d = c - (_NBUF - 1)
                b = d % _NBUF
                off = base + d * _CHUNK
                cp_v[b].wait()
                pltpu.sync_copy(rows_v[b], emb_out.at[pl.ds(off, _CHUNK)])
                cp_w[b].wait()
                pltpu.sync_copy(wrows_v[b], w_out.at[pl.ds(off, _CHUNK)])

    return sc_gather


def _tc_body(emb_ref, wv_ref, s_ref, w0_ref, b0_ref, w1_ref, b1_ref,
             w2_ref, b2_ref, w3_ref, b3_ref, wout_ref, bout_ref, out_ref):
    e = emb_ref[...]
    s = s_ref[...]
    sv = jnp.dot(e, s, preferred_element_type=jnp.float32)
    sv2 = jnp.dot(e * e, s, preferred_element_type=jnp.float32)
    fm = 0.5 * jnp.sum(sv * sv - sv2, axis=1, keepdims=True)
    wsum = jnp.sum(wv_ref[...], axis=1, keepdims=True)
    h = jnp.maximum(
        jnp.dot(e, w0_ref[...], preferred_element_type=jnp.float32)
        + b0_ref[...], 0.0)
    h = jnp.maximum(
        jnp.dot(h, w1_ref[...], preferred_element_type=jnp.float32)
        + b1_ref[...], 0.0)
    h = jnp.maximum(
        jnp.dot(h, w2_ref[...], preferred_element_type=jnp.float32)
        + b2_ref[...], 0.0)
    h = jnp.maximum(
        jnp.dot(h, w3_ref[...], preferred_element_type=jnp.float32)
        + b3_ref[...], 0.0)
    out_ref[...] = fm + wsum + h * wout_ref[0, 0] + bout_ref[0, 0]


def _tc_deepfm(emb, wv, sel, W0, b0, W1, b1, W2, b2, W3, b3, W_out, b_out,
               block_b=512):
    batch, fk = emb.shape
    grid = (batch // block_b,)
    const = lambda i: (0, 0)
    return pl.pallas_call(
        _tc_body,
        grid=grid,
        in_specs=[
            pl.BlockSpec((block_b, fk), lambda i: (i, 0)),
            pl.BlockSpec((block_b, _FIELDS), lambda i: (i, 0)),
            pl.BlockSpec(sel.shape, const),
            pl.BlockSpec(W0.shape, const),
            pl.BlockSpec(b0.shape, const),
            pl.BlockSpec(W1.shape, const),
            pl.BlockSpec(b1.shape, const),
            pl.BlockSpec(W2.shape, const),
            pl.BlockSpec(b2.shape, const),
            pl.BlockSpec(W3.shape, const),
            pl.BlockSpec(b3.shape, const),
            pl.BlockSpec(W_out.shape, const),
            pl.BlockSpec(b_out.shape, const),
        ],
        out_specs=pl.BlockSpec((block_b, 1), lambda i: (i, 0)),
        out_shape=jax.ShapeDtypeStruct((batch, 1), jnp.float32),
    )(emb, wv, sel, W0, b0, W1, b1, W2, b2, W3, b3, W_out, b_out)


def kernel(feature, v_table, w_table, W0, b0, W1, b1, W2, b2, W3, b3,
           W_out, b_out):
    batch, fields = feature.shape
    k_dim = v_table.shape[1]
    n_idx = batch * fields
    idx = feature.reshape(n_idx // _CHUNK, _CHUNK).astype(jnp.int32)

    emb_rows, w_rows = _make_sc_gather(n_idx, k_dim)(
        idx, v_table, w_table.reshape(-1))
    emb = emb_rows.reshape(batch, fields * k_dim)
    wv = w_rows.reshape(batch, fields)

    sel = jnp.tile(jnp.eye(k_dim, dtype=jnp.float32), (fields, 1))

    return _tc_deepfm(emb, wv, sel,
                      W0, b0.reshape(1, -1), W1, b1.reshape(1, -1),
                      W2, b2.reshape(1, -1), W3, b3.reshape(1, -1),
                      W_out, b_out.reshape(1, 1))


# speedup vs baseline: 1.5220x; 1.5220x over previous
import functools

import jax
import jax.numpy as jnp
from jax import lax
from jax.experimental import pallas as pl
from jax.experimental.pallas import tpu as pltpu
from jax.experimental.pallas import tpu_sc as plsc

_FIELDS = 26
_K = 32
_NC = 2
_NS = 16


_CHUNK = 128
_NBUF = 4


def _make_sc_gather(n_idx, k_dim):
    nw = _NC * _NS
    assert n_idx % nw == 0
    b_per_w = n_idx // nw
    assert b_per_w % _CHUNK == 0
    n_chunks = b_per_w // _CHUNK
    mesh = plsc.VectorSubcoreMesh(core_axis_name="c", subcore_axis_name="s")

    @functools.partial(
        pl.kernel,
        mesh=mesh,
        compiler_params=pltpu.CompilerParams(use_tc_tiling_on_sc=False),
        out_type=[
            jax.ShapeDtypeStruct((n_idx, k_dim), jnp.float32),
            jax.ShapeDtypeStruct((n_idx,), jnp.float32),
        ],
        scratch_types=[
            pltpu.VMEM((n_chunks, _CHUNK), jnp.int32),
            [pltpu.VMEM((_CHUNK, 128), jnp.float32) for _ in range(_NBUF)],
            [pltpu.VMEM((_CHUNK,), jnp.float32) for _ in range(_NBUF)],
            [pltpu.SemaphoreType.DMA for _ in range(_NBUF)],
            [pltpu.SemaphoreType.DMA for _ in range(_NBUF)],
        ],
    )
    def sc_gather(idx_hbm, v_hbm, w_hbm, emb_out, w_out,
                  idx_v, rows_v, wrows_v, sem_v, sem_w):
        wid = lax.axis_index("s") * _NC + lax.axis_index("c")
        base = wid * b_per_w
        pltpu.sync_copy(
            idx_hbm.at[pl.ds(wid * n_chunks, n_chunks)], idx_v)
        cp_v = [None] * _NBUF
        cp_w = [None] * _NBUF
        for c in range(n_chunks + _NBUF - 1):
            if c < n_chunks:
                b = c % _NBUF
                cp_v[b] = pltpu.async_copy(
                    v_hbm.at[idx_v.at[c]], rows_v[b], sem_v[b])
                cp_w[b] = pltpu.async_copy(
                    w_hbm.at[idx_v.at[c]], wrows_v[b], sem_w[b])
            if c >= _NBUF - 1:
                d = c - (_NBUF - 1)
                b = d % _NBUF
                off = base + d * _CHUNK
                cp_v[b].wait()
                pltpu.sync_copy(rows_v[b].at[:, pl.ds(0, k_dim)],
                                emb_out.at[pl.ds(off, _CHUNK)])
                cp_w[b].wait()
                pltpu.sync_copy(wrows_v[b], w_out.at[pl.ds(off, _CHUNK)])

    return sc_gather


_TRC = 8192


def _transpose_body(vt_ref, out_ref):
    xt = jnp.transpose(vt_ref[...])
    out_ref[...] = jnp.concatenate(
        [xt, jnp.zeros((xt.shape[0], 128 - xt.shape[1]), jnp.float32)],
        axis=1)


def _transpose_table(vt):
    k_dim, v_rows = vt.shape
    grid = (pl.cdiv(v_rows, _TRC),)
    return pl.pallas_call(
        _transpose_body,
        grid=grid,
        in_specs=[pl.BlockSpec((k_dim, _TRC), lambda c: (0, c))],
        out_specs=pl.BlockSpec((_TRC, 128), lambda c: (c, 0)),
        out_shape=jax.ShapeDtypeStruct((v_rows, 128), jnp.float32),
    )(vt)


def _tc_body(emb_ref, wv_ref, s_ref, w0_ref, b0_ref, w1_ref, b1_ref,
             w2_ref, b2_ref, w3_ref, b3_ref, wout_ref, bout_ref, out_ref):
    e = emb_ref[...]
    s = s_ref[...]
    sv = jnp.dot(e, s, preferred_element_type=jnp.float32)
    sv2 = jnp.dot(e * e, s, preferred_element_type=jnp.float32)
    fm = 0.5 * jnp.sum(sv * sv - sv2, axis=1, keepdims=True)
    wsum = jnp.sum(wv_ref[...], axis=1, keepdims=True)
    h = jnp.maximum(
        jnp.dot(e, w0_ref[...], preferred_element_type=jnp.float32)
        + b0_ref[...], 0.0)
    h = jnp.maximum(
        jnp.dot(h, w1_ref[...], preferred_element_type=jnp.float32)
        + b1_ref[...], 0.0)
    h = jnp.maximum(
        jnp.dot(h, w2_ref[...], preferred_element_type=jnp.float32)
        + b2_ref[...], 0.0)
    h = jnp.maximum(
        jnp.dot(h, w3_ref[...], preferred_element_type=jnp.float32)
        + b3_ref[...], 0.0)
    out_ref[...] = fm + wsum + h * wout_ref[0, 0] + bout_ref[0, 0]


def _tc_deepfm(emb, wv, sel, W0, b0, W1, b1, W2, b2, W3, b3, W_out, b_out,
               block_b=512):
    batch, fk = emb.shape
    grid = (batch // block_b,)
    const = lambda i: (0, 0)
    return pl.pallas_call(
        _tc_body,
        grid=grid,
        in_specs=[
            pl.BlockSpec((block_b, fk), lambda i: (i, 0)),
            pl.BlockSpec((block_b, _FIELDS), lambda i: (i, 0)),
            pl.BlockSpec(sel.shape, const),
            pl.BlockSpec(W0.shape, const),
            pl.BlockSpec(b0.shape, const),
            pl.BlockSpec(W1.shape, const),
            pl.BlockSpec(b1.shape, const),
            pl.BlockSpec(W2.shape, const),
            pl.BlockSpec(b2.shape, const),
            pl.BlockSpec(W3.shape, const),
            pl.BlockSpec(b3.shape, const),
            pl.BlockSpec(W_out.shape, const),
            pl.BlockSpec(b_out.shape, const),
        ],
        out_specs=pl.BlockSpec((block_b, 1), lambda i: (i, 0)),
        out_shape=jax.ShapeDtypeStruct((batch, 1), jnp.float32),
    )(emb, wv, sel, W0, b0, W1, b1, W2, b2, W3, b3, W_out, b_out)


def kernel(feature, v_table, w_table, W0, b0, W1, b1, W2, b2, W3, b3,
           W_out, b_out):
    batch, fields = feature.shape
    k_dim = v_table.shape[1]
    n_idx = batch * fields
    idx = feature.reshape(n_idx // _CHUNK, _CHUNK).astype(jnp.int32)

    v_dense = _transpose_table(v_table.T)

    emb_rows, w_rows = _make_sc_gather(n_idx, k_dim)(
        idx, v_dense, w_table.reshape(-1))
    emb = emb_rows.reshape(batch, fields * k_dim)
    wv = w_rows.reshape(batch, fields)

    sel = jnp.tile(jnp.eye(k_dim, dtype=jnp.float32), (fields, 1))

    return _tc_deepfm(emb, wv, sel,
                      W0, b0.reshape(1, -1), W1, b1.reshape(1, -1),
                      W2, b2.reshape(1, -1), W3, b3.reshape(1, -1),
                      W_out, b_out.reshape(1, 1))
